# baseline (speedup 1.0000x reference)
import functools

import jax
import jax.numpy as jnp
from jax import lax
from jax.experimental import pallas as pl
from jax.experimental.pallas import tpu as pltpu
from jax.experimental.pallas import tpu_sc as plsc

NW = 32
L = 16
CHUNK = 128
BV = 2048


def _tc_normalize_body(emb_t_ref, out_ref):
    x = emb_t_ref[...]
    s = jnp.sum(x * x, axis=0)
    y = jax.lax.rsqrt(s)
    out_ref[...] = (x * y[None, :]).T


def _normalized_table(emb_t, vocab, d_dim):
    grid = (vocab + BV - 1) // BV
    return pl.pallas_call(
        _tc_normalize_body,
        grid=(grid,),
        in_specs=[pl.BlockSpec((d_dim, BV), lambda i: (0, i))],
        out_specs=pl.BlockSpec((BV, d_dim), lambda i: (i, 0)),
        out_shape=jax.ShapeDtypeStruct((vocab, d_dim), jnp.float32),
    )(emb_t)


def kernel(indices, embeddings):
    b_dim, f_dim = indices.shape
    vocab, d_dim = embeddings.shape
    flat_b = indices.size
    assert flat_b % (NW * CHUNK) == 0
    b_per_w = flat_b // NW
    n_chunks = b_per_w // CHUNK

    table_n = _normalized_table(embeddings.T, vocab, d_dim)

    idx3 = indices.T.astype(jnp.int32).reshape(NW, n_chunks, CHUNK)

    mesh = plsc.VectorSubcoreMesh(core_axis_name="c", subcore_axis_name="s")

    @functools.partial(
        pl.kernel,
        mesh=mesh,
        compiler_params=pltpu.CompilerParams(
            needs_layout_passes=False, use_tc_tiling_on_sc=False),
        out_type=jax.ShapeDtypeStruct((flat_b, d_dim), jnp.float32),
        scratch_types=[
            pltpu.VMEM((n_chunks, CHUNK), jnp.int32),
            pltpu.VMEM((CHUNK, d_dim), jnp.float32),
            pltpu.SemaphoreType.DMA,
        ],
    )
    def run(table_hbm, idx_hbm, out_hbm, idx_v, buf, sem):
        wid = lax.axis_index("s") * 2 + lax.axis_index("c")
        pltpu.sync_copy(idx_hbm.at[wid], idx_v)

        def chunk_body(c, carry):
            pltpu.async_copy(table_hbm.at[idx_v.at[c]], buf, sem).wait()
            base = wid * b_per_w + c * CHUNK
            pltpu.sync_copy(buf, out_hbm.at[pl.ds(base, CHUNK)])
            return carry

        lax.fori_loop(0, n_chunks, chunk_body, 0)

    out = run(table_n, idx3)
    return out.reshape(f_dim, b_dim, d_dim).transpose(1, 0, 2)
